# Initial kernel scaffold; baseline (speedup 1.0000x reference)
import jax
import jax.numpy as jnp
from jax.experimental import pallas as pl


def kernel(x, p2, masks, rand_coords, rand_extra, fc1_w, fc1_b, fc2_w, fc2_b, fc3_w, fc3_b, pred_w, pred_b):
    raise NotImplementedError("write your pallas kernel here")


# speedup vs baseline: 1.0119x; 1.0119x over previous
import functools
import jax
import jax.numpy as jnp
from jax import lax
from jax.experimental import pallas as pl
from jax.experimental.pallas import tpu as pltpu
from jax.experimental.pallas import tpu_sc as plsc


def _head_body(c_ref, f_ref, wc1, wf1, b1, wc2, wf2, b2, wc3, wf3, b3,
               wcp, wfp, bp, o_ref):
    c = c_ref[...]
    f = f_ref[...]
    dot = functools.partial(jnp.dot, preferred_element_type=jnp.float32)
    h = jax.nn.relu(dot(c, wc1[...]) + dot(f, wf1[...]) + b1[...])
    h = jax.nn.relu(dot(h, wf2[...]) + dot(c, wc2[...]) + b2[...])
    h = jax.nn.relu(dot(h, wf3[...]) + dot(c, wc3[...]) + b3[...])
    o_ref[...] = dot(h, wfp[...]) + dot(c, wcp[...]) + bp[...]


def _point_head(coarse, fine, fc1_w, fc1_b, fc2_w, fc2_b, fc3_w, fc3_b,
                pred_w, pred_b):
    N, P, CC = coarse.shape
    BP = 2048
    wc1 = fc1_w[:, :CC].T
    wf1 = fc1_w[:, CC:].T
    wf2, wc2 = fc2_w[:, :256].T, fc2_w[:, 256:].T
    wf3, wc3 = fc3_w[:, :256].T, fc3_w[:, 256:].T
    wfp, wcp = pred_w[:, :256].T, pred_w[:, 256:].T
    b1 = fc1_b[None, :]
    b2 = fc2_b[None, :]
    b3 = fc3_b[None, :]
    bp = pred_b[None, :]

    grid = (N, P // BP)
    wspec = lambda shape: pl.BlockSpec(shape, lambda n, p: (0, 0))
    out = pl.pallas_call(
        _head_body,
        grid=grid,
        in_specs=[
            pl.BlockSpec((1, BP, CC), lambda n, p: (n, p, 0)),
            pl.BlockSpec((1, BP, 256), lambda n, p: (n, p, 0)),
            wspec(wc1.shape), wspec(wf1.shape), wspec(b1.shape),
            wspec(wc2.shape), wspec(wf2.shape), wspec(b2.shape),
            wspec(wc3.shape), wspec(wf3.shape), wspec(b3.shape),
            wspec(wcp.shape), wspec(wfp.shape), wspec(bp.shape),
        ],
        out_specs=pl.BlockSpec((1, BP, CC), lambda n, p: (n, p, 0)),
        out_shape=jax.ShapeDtypeStruct((N, P, CC), jnp.float32),
    )(coarse, fine, wc1, wf1, b1, wc2, wf2, b2, wc3, wf3, b3, wcp, wfp, bp)
    return out


def _grid_sample(img, pc):
    N, C, H, W = img.shape
    g = 2.0 * pc - 1.0
    x = ((g[..., 0] + 1.0) * W - 1.0) / 2.0
    y = ((g[..., 1] + 1.0) * H - 1.0) / 2.0
    x0 = jnp.floor(x)
    y0 = jnp.floor(y)
    wx1 = x - x0
    wy1 = y - y0
    flat = img.reshape(N, C, H * W)

    def gather(ix, iy):
        valid = ((ix >= 0) & (ix <= W - 1) & (iy >= 0) & (iy <= H - 1)).astype(img.dtype)
        ixc = jnp.clip(ix, 0, W - 1).astype(jnp.int32)
        iyc = jnp.clip(iy, 0, H - 1).astype(jnp.int32)
        idx = iyc * W + ixc
        vals = jnp.take_along_axis(flat, idx[:, None, :], axis=2)
        return vals * valid[:, None, :]

    out = (gather(x0, y0) * ((1.0 - wx1) * (1.0 - wy1))[:, None, :]
           + gather(x0 + 1.0, y0) * (wx1 * (1.0 - wy1))[:, None, :]
           + gather(x0, y0 + 1.0) * ((1.0 - wx1) * wy1)[:, None, :]
           + gather(x0 + 1.0, y0 + 1.0) * (wx1 * wy1)[:, None, :])
    return out


def kernel(x, p2, masks, rand_coords, rand_extra, fc1_w, fc1_b, fc2_w, fc2_b,
           fc3_w, fc3_b, pred_w, pred_b):
    num_points = masks.shape[-1] ** 2
    num_uncertain = int(0.75 * num_points)
    point_logits = _grid_sample(masks, rand_coords)
    t = lax.top_k(jnp.transpose(point_logits, (0, 2, 1)), 2)[0]
    unc = t[..., 1] - t[..., 0]
    idx = lax.top_k(unc, num_uncertain)[1]
    sel = jnp.take_along_axis(rand_coords, idx[:, :, None], axis=1)
    point_coords = jnp.concatenate([sel, rand_extra], axis=1)
    coarse = _grid_sample(masks, point_coords)
    fine = _grid_sample(p2, point_coords)
    rend = _point_head(jnp.transpose(coarse, (0, 2, 1)),
                       jnp.transpose(fine, (0, 2, 1)),
                       fc1_w, fc1_b, fc2_w, fc2_b, fc3_w, fc3_b,
                       pred_w, pred_b)
    return (jnp.transpose(rend, (0, 2, 1)), point_coords)


# speedup vs baseline: 10.2025x; 10.0828x over previous
import functools
import jax
import jax.numpy as jnp
from jax import lax
from jax.experimental import pallas as pl
from jax.experimental.pallas import tpu as pltpu
from jax.experimental.pallas import tpu_sc as plsc

_NC, _NS, _L = 2, 16, 16
_NPTS = 12288
_K = 3072
_P = 4096
_PT = _P // 4
_CHUNK = _NPTS // _L


def _wid():
    return lax.axis_index("s") * _NC + lax.axis_index("c")


def _corners(xg, yg, H, W):
    ione = jnp.ones((_L,), jnp.int32)
    izero = jnp.zeros((_L,), jnp.int32)
    fone = jnp.full((_L,), 1.0, jnp.float32)
    fzero = jnp.zeros((_L,), jnp.float32)
    gx = 2.0 * xg - 1.0
    gy = 2.0 * yg - 1.0
    x = ((gx + 1.0) * jnp.float32(W) - 1.0) * 0.5
    y = ((gy + 1.0) * jnp.float32(H) - 1.0) * 0.5
    xt = x.astype(jnp.int32)
    x0i = xt - jnp.where(x < xt.astype(jnp.float32), ione, izero)
    yt = y.astype(jnp.int32)
    y0i = yt - jnp.where(y < yt.astype(jnp.float32), ione, izero)
    wx1 = x - x0i.astype(jnp.float32)
    wy1 = y - y0i.astype(jnp.float32)
    wx0 = 1.0 - wx1
    wy0 = 1.0 - wy1
    out = []
    for (dx, dy, wx, wy) in ((0, 0, wx0, wy0), (1, 0, wx1, wy0),
                             (0, 1, wx0, wy1), (1, 1, wx1, wy1)):
        ix = x0i + dx
        iy = y0i + dy
        valid = ((ix >= 0) & (ix <= W - 1) & (iy >= 0) & (iy <= H - 1))
        vf = jnp.where(valid, fone, fzero)
        w = wx * wy
        ixc = jnp.minimum(jnp.maximum(ix, izero), W - 1)
        iyc = jnp.minimum(jnp.maximum(iy, izero), H - 1)
        out.append((iyc * W + ixc, vf, w))
    return out


def _unc_body(mask_hbm, cx_hbm, cy_hbm, unc_hbm, mask_v, cx_v, cy_v, unc_v):
    wid = _wid()
    img = wid // 4
    base = (wid % 4) * (_NPTS // 4)
    pltpu.sync_copy(mask_hbm.at[img], mask_v)
    pltpu.sync_copy(cx_hbm.at[img, pl.ds(base, _NPTS // 4)], cx_v)
    pltpu.sync_copy(cy_hbm.at[img, pl.ds(base, _NPTS // 4)], cy_v)

    def step(t, _):
        xg = cx_v[pl.ds(t * _L, _L)]
        yg = cy_v[pl.ds(t * _L, _L)]
        cs = _corners(xg, yg, 64, 64)
        acc = [jnp.zeros((_L,), jnp.float32) for _ in range(20)]
        for (row, vf, w) in cs:
            rowb = row * 20
            for c in range(20):
                g = plsc.load_gather(mask_v, [rowb + c])
                acc[c] = acc[c] + (g * vf) * w
        m1 = acc[0]
        m2 = jnp.full((_L,), -jnp.inf, jnp.float32)
        for c in range(1, 20):
            v = acc[c]
            gt = v > m1
            m2 = jnp.where(gt, m1, jnp.maximum(m2, v))
            m1 = jnp.maximum(m1, v)
        unc_v[pl.ds(t * _L, _L)] = m2 - m1
        return _

    lax.fori_loop(0, (_NPTS // 4) // _L, step, 0)
    pltpu.sync_copy(unc_v, unc_hbm.at[img, pl.ds(base, _NPTS // 4)])


def _sample_unc(mask_t, cx, cy):
    mesh = plsc.VectorSubcoreMesh(core_axis_name="c", subcore_axis_name="s")
    f = pl.kernel(
        _unc_body,
        out_type=jax.ShapeDtypeStruct((8, _NPTS), jnp.float32),
        mesh=mesh,
        compiler_params=pltpu.CompilerParams(needs_layout_passes=False),
        scratch_types=[
            pltpu.VMEM((4096 * 20,), jnp.float32),
            pltpu.VMEM((_NPTS // 4,), jnp.float32),
            pltpu.VMEM((_NPTS // 4,), jnp.float32),
            pltpu.VMEM((_NPTS // 4,), jnp.float32),
        ],
    )
    return f(mask_t, cx, cy)


def _sort_body(unc_hbm, cx_hbm, cy_hbm, scx_hbm, scy_hbm,
               key_a, key_b, idx_a, idx_b, cnt, cx_v, cy_v, sx_v, sy_v):
    wid = _wid()
    lanes = lax.iota(jnp.int32, _L)
    lb = lanes * _CHUNK

    @pl.when(wid < 8)
    def _():
        img = wid
        pltpu.sync_copy(unc_hbm.at[img], key_a)
        pltpu.sync_copy(cx_hbm.at[img], cx_v)
        pltpu.sync_copy(cy_hbm.at[img], cy_v)

        minint = jnp.full((_L,), -2147483648, jnp.int32)
        izero = jnp.full((_L,), 0, jnp.int32)

        def keyify(t, c):
            u = key_a[pl.ds(t * _L, _L)]
            mono = jnp.where(u >= izero, u ^ minint, ~u)
            key_a[pl.ds(t * _L, _L)] = ~mono
            return c
        lax.fori_loop(0, _NPTS // _L, keyify, 0)

        def one_pass(src_k, src_i, dst_k, dst_i, shift, first, last):
            def zero(j, c):
                cnt[pl.ds(j * _L, _L)] = jnp.zeros((_L,), jnp.int32)
                return c
            lax.fori_loop(0, 256, zero, 0)

            shv = jnp.full((_L,), shift, jnp.int32)
            m255 = jnp.full((_L,), 255, jnp.int32)

            def hist(t, c):
                k = plsc.load_gather(src_k, [lb + t])
                d = lax.shift_right_logical(k, shv) & m255
                a = d * _L + lanes
                cv = plsc.load_gather(cnt, [a])
                plsc.store_scatter(cnt, [a], cv + 1)
                return c
            lax.fori_loop(0, _CHUNK, hist, 0)

            def scan(j, carry):
                v = cnt[pl.ds(j * _L, _L)]
                inc = plsc.cumsum(v)
                cnt[pl.ds(j * _L, _L)] = carry + (inc - v)
                return carry + jnp.sum(v)
            lax.fori_loop(0, 256, scan, jnp.int32(0))

            def perm(t, c):
                k = plsc.load_gather(src_k, [lb + t])
                if first:
                    v = lb + t
                else:
                    v = plsc.load_gather(src_i, [lb + t])
                d = lax.shift_right_logical(k, shv) & m255
                a = d * _L + lanes
                pos = plsc.load_gather(cnt, [a])
                plsc.store_scatter(cnt, [a], pos + 1)
                if last:
                    m = pos < _K
                    plsc.store_scatter(sx_v, [pos],
                                       plsc.load_gather(cx_v, [v]), mask=m)
                    plsc.store_scatter(sy_v, [pos],
                                       plsc.load_gather(cy_v, [v]), mask=m)
                else:
                    plsc.store_scatter(dst_k, [pos], k)
                    plsc.store_scatter(dst_i, [pos], v)
                return c
            lax.fori_loop(0, _CHUNK, perm, 0)

        one_pass(key_a, idx_a, key_b, idx_b, 0, True, False)
        one_pass(key_b, idx_b, key_a, idx_a, 8, False, False)
        one_pass(key_a, idx_a, key_b, idx_b, 16, False, False)
        one_pass(key_b, idx_b, key_a, idx_a, 24, False, True)

        pltpu.sync_copy(sx_v, scx_hbm.at[img])
        pltpu.sync_copy(sy_v, scy_hbm.at[img])


def _topk_coords(unc, cx, cy):
    unc = lax.bitcast_convert_type(unc, jnp.int32)
    mesh = plsc.VectorSubcoreMesh(core_axis_name="c", subcore_axis_name="s")
    f = pl.kernel(
        _sort_body,
        out_type=(jax.ShapeDtypeStruct((8, _K), jnp.float32),
                  jax.ShapeDtypeStruct((8, _K), jnp.float32)),
        mesh=mesh,
        compiler_params=pltpu.CompilerParams(needs_layout_passes=False),
        scratch_types=[
            pltpu.VMEM((_NPTS,), jnp.int32),
            pltpu.VMEM((_NPTS,), jnp.int32),
            pltpu.VMEM((_NPTS,), jnp.int32),
            pltpu.VMEM((_NPTS,), jnp.int32),
            pltpu.VMEM((256 * _L,), jnp.int32),
            pltpu.VMEM((_NPTS,), jnp.float32),
            pltpu.VMEM((_NPTS,), jnp.float32),
            pltpu.VMEM((_K,), jnp.float32),
            pltpu.VMEM((_K,), jnp.float32),
        ],
    )
    return f(unc, cx, cy)


def _coarse_body(mask_hbm, px_hbm, py_hbm, co_hbm, mask_v, px_v, py_v, co_v):
    wid = _wid()
    img = wid // 4
    pbase = (wid % 4) * _PT
    lanes = lax.iota(jnp.int32, _L)
    pltpu.sync_copy(mask_hbm.at[img], mask_v)
    pltpu.sync_copy(px_hbm.at[img, pl.ds(pbase, _PT)], px_v)
    pltpu.sync_copy(py_hbm.at[img, pl.ds(pbase, _PT)], py_v)

    def step(t, _):
        xg = px_v[pl.ds(t * _L, _L)]
        yg = py_v[pl.ds(t * _L, _L)]
        cs = _corners(xg, yg, 64, 64)
        pb20 = (t * _L + lanes) * 20
        for c in range(20):
            acc = jnp.zeros((_L,), jnp.float32)
            for (row, vf, w) in cs:
                g = plsc.load_gather(mask_v, [row * 20 + c])
                acc = acc + (g * vf) * w
            plsc.store_scatter(co_v, [pb20 + c], acc)
        return _

    lax.fori_loop(0, _PT // _L, step, 0)
    pltpu.sync_copy(co_v, co_hbm.at[img, pl.ds(pbase * 20, _PT * 20)])


def _sample_coarse(mask_t, px, py):
    mesh = plsc.VectorSubcoreMesh(core_axis_name="c", subcore_axis_name="s")
    f = pl.kernel(
        _coarse_body,
        out_type=jax.ShapeDtypeStruct((8, _P * 20), jnp.float32),
        mesh=mesh,
        compiler_params=pltpu.CompilerParams(needs_layout_passes=False),
        scratch_types=[
            pltpu.VMEM((4096 * 20,), jnp.float32),
            pltpu.VMEM((_PT,), jnp.float32),
            pltpu.VMEM((_PT,), jnp.float32),
            pltpu.VMEM((_PT * 20,), jnp.float32),
        ],
    )
    return f(mask_t, px, py)


def _fine_body(p2_hbm, px_hbm, py_hbm, fi_hbm, px_v, py_v, idx_v,
               rows_v, out_v, sem):
    wid = _wid()
    img = wid // 4
    pbase = (wid % 4) * _PT
    pltpu.sync_copy(px_hbm.at[img, pl.ds(pbase, _PT)], px_v)
    pltpu.sync_copy(py_hbm.at[img, pl.ds(pbase, _PT)], py_v)
    gbase = img * 16384
    gdn = lax.GatherDimensionNumbers(offset_dims=(), collapsed_slice_dims=(0,),
                                     start_index_map=(0,))

    def step(t, _):
        xg = px_v[pl.ds(t * _L, _L)]
        yg = py_v[pl.ds(t * _L, _L)]
        cs = _corners(xg, yg, 128, 128)
        wv = []
        for k, (row, vf, w) in enumerate(cs):
            idx_v[pl.ds(k * _L, _L)] = gbase + row
            wv.append(vf * w)
        pltpu.async_copy(p2_hbm.at[idx_v], rows_v, sem).wait()
        for p in range(_L):
            pidx = jnp.full((_L, 1), p, jnp.int32)
            wk = [lax.gather(wv[k], pidx, gdn, (1,),
                             mode=lax.GatherScatterMode.PROMISE_IN_BOUNDS)
                  for k in range(4)]
            for cg in range(256 // _L):
                acc = jnp.zeros((_L,), jnp.float32)
                for k in range(4):
                    r = rows_v[k * _L + p, pl.ds(cg * _L, _L)]
                    acc = acc + r * wk[k]
                out_v[pl.ds(p * 256 + cg * _L, _L)] = acc
        pltpu.sync_copy(out_v, fi_hbm.at[img, pl.ds((pbase + t * _L) * 256,
                                                    _L * 256)])
        return _

    lax.fori_loop(0, _PT // _L, step, 0)


def _sample_fine(p2f, px, py):
    mesh = plsc.VectorSubcoreMesh(core_axis_name="c", subcore_axis_name="s")
    f = pl.kernel(
        _fine_body,
        out_type=jax.ShapeDtypeStruct((8, _P * 256), jnp.float32),
        mesh=mesh,
        compiler_params=pltpu.CompilerParams(needs_layout_passes=False),
        scratch_types=[
            pltpu.VMEM((_PT,), jnp.float32),
            pltpu.VMEM((_PT,), jnp.float32),
            pltpu.VMEM((4 * _L,), jnp.int32),
            pltpu.VMEM((4 * _L, 256), jnp.float32),
            pltpu.VMEM((_L * 256,), jnp.float32),
            pltpu.SemaphoreType.DMA,
        ],
    )
    return f(p2f, px, py)


def _head_body(c_ref, f_ref, wc1, wf1, b1, wc2, wf2, b2, wc3, wf3, b3,
               wcp, wfp, bp, o_ref):
    c = c_ref[...]
    f = f_ref[...]
    dot = functools.partial(jnp.dot, preferred_element_type=jnp.float32)
    h = jax.nn.relu(dot(c, wc1[...]) + dot(f, wf1[...]) + b1[...])
    h = jax.nn.relu(dot(h, wf2[...]) + dot(c, wc2[...]) + b2[...])
    h = jax.nn.relu(dot(h, wf3[...]) + dot(c, wc3[...]) + b3[...])
    o_ref[...] = dot(h, wfp[...]) + dot(c, wcp[...]) + bp[...]


def _point_head(coarse, fine, fc1_w, fc1_b, fc2_w, fc2_b, fc3_w, fc3_b,
                pred_w, pred_b):
    N, P, CC = coarse.shape
    BP = 2048
    wc1 = fc1_w[:, :CC].T
    wf1 = fc1_w[:, CC:].T
    wf2, wc2 = fc2_w[:, :256].T, fc2_w[:, 256:].T
    wf3, wc3 = fc3_w[:, :256].T, fc3_w[:, 256:].T
    wfp, wcp = pred_w[:, :256].T, pred_w[:, 256:].T
    b1, b2, b3, bp = (fc1_b[None, :], fc2_b[None, :], fc3_b[None, :],
                      pred_b[None, :])
    grid = (N, P // BP)
    wspec = lambda shape: pl.BlockSpec(shape, lambda n, p: (0, 0))
    return pl.pallas_call(
        _head_body,
        grid=grid,
        in_specs=[
            pl.BlockSpec((1, BP, CC), lambda n, p: (n, p, 0)),
            pl.BlockSpec((1, BP, 256), lambda n, p: (n, p, 0)),
            wspec(wc1.shape), wspec(wf1.shape), wspec(b1.shape),
            wspec(wc2.shape), wspec(wf2.shape), wspec(b2.shape),
            wspec(wc3.shape), wspec(wf3.shape), wspec(b3.shape),
            wspec(wcp.shape), wspec(wfp.shape), wspec(bp.shape),
        ],
        out_specs=pl.BlockSpec((1, BP, CC), lambda n, p: (n, p, 0)),
        out_shape=jax.ShapeDtypeStruct((N, P, CC), jnp.float32),
    )(coarse, fine, wc1, wf1, b1, wc2, wf2, b2, wc3, wf3, b3, wcp, wfp, bp)


def kernel(x, p2, masks, rand_coords, rand_extra, fc1_w, fc1_b, fc2_w, fc2_b,
           fc3_w, fc3_b, pred_w, pred_b):
    N = masks.shape[0]
    mask_t = jnp.transpose(masks.reshape(N, 20, 4096), (0, 2, 1))
    mask_t = mask_t.reshape(N, 4096 * 20)
    p2f = jnp.transpose(p2.reshape(N, 256, 16384), (0, 2, 1))
    p2f = p2f.reshape(N * 16384, 256)
    cx = rand_coords[..., 0]
    cy = rand_coords[..., 1]

    unc = _sample_unc(mask_t, cx, cy)
    sx, sy = _topk_coords(unc, cx, cy)
    px = jnp.concatenate([sx, rand_extra[..., 0]], axis=1)
    py = jnp.concatenate([sy, rand_extra[..., 1]], axis=1)
    point_coords = jnp.stack([px, py], axis=-1)

    coarse = _sample_coarse(mask_t, px, py).reshape(N, _P, 20)
    fine = _sample_fine(p2f, px, py).reshape(N, _P, 256)
    rend = _point_head(coarse, fine, fc1_w, fc1_b, fc2_w, fc2_b,
                       fc3_w, fc3_b, pred_w, pred_b)
    return (jnp.transpose(rend, (0, 2, 1)), point_coords)


# speedup vs baseline: 11.3716x; 1.1146x over previous
import functools
import jax
import jax.numpy as jnp
from jax import lax
from jax.experimental import pallas as pl
from jax.experimental.pallas import tpu as pltpu
from jax.experimental.pallas import tpu_sc as plsc

_NC, _NS, _L = 2, 16, 16
_NPTS = 12288
_K = 3072
_P = 4096
_PT = _P // 4
_CHUNK = _NPTS // _L


def _wid():
    return lax.axis_index("s") * _NC + lax.axis_index("c")


def _corners(xg, yg, H, W):
    ione = jnp.ones((_L,), jnp.int32)
    izero = jnp.zeros((_L,), jnp.int32)
    fone = jnp.full((_L,), 1.0, jnp.float32)
    fzero = jnp.zeros((_L,), jnp.float32)
    gx = 2.0 * xg - 1.0
    gy = 2.0 * yg - 1.0
    x = ((gx + 1.0) * jnp.float32(W) - 1.0) * 0.5
    y = ((gy + 1.0) * jnp.float32(H) - 1.0) * 0.5
    xt = x.astype(jnp.int32)
    x0i = xt - jnp.where(x < xt.astype(jnp.float32), ione, izero)
    yt = y.astype(jnp.int32)
    y0i = yt - jnp.where(y < yt.astype(jnp.float32), ione, izero)
    wx1 = x - x0i.astype(jnp.float32)
    wy1 = y - y0i.astype(jnp.float32)
    wx0 = 1.0 - wx1
    wy0 = 1.0 - wy1
    out = []
    for (dx, dy, wx, wy) in ((0, 0, wx0, wy0), (1, 0, wx1, wy0),
                             (0, 1, wx0, wy1), (1, 1, wx1, wy1)):
        ix = x0i + dx
        iy = y0i + dy
        valid = ((ix >= 0) & (ix <= W - 1) & (iy >= 0) & (iy <= H - 1))
        vf = jnp.where(valid, fone, fzero)
        w = wx * wy
        ixc = jnp.minimum(jnp.maximum(ix, izero), W - 1)
        iyc = jnp.minimum(jnp.maximum(iy, izero), H - 1)
        out.append((iyc * W + ixc, vf, w))
    return out


def _unc_body(mask_hbm, cx_hbm, cy_hbm, unc_hbm, mask_v, cx_v, cy_v, unc_v):
    wid = _wid()
    img = wid // 4
    base = (wid % 4) * (_NPTS // 4)
    pltpu.sync_copy(mask_hbm.at[img], mask_v)
    pltpu.sync_copy(cx_hbm.at[img, pl.ds(base, _NPTS // 4)], cx_v)
    pltpu.sync_copy(cy_hbm.at[img, pl.ds(base, _NPTS // 4)], cy_v)

    def step(t, _):
        xg = cx_v[pl.ds(t * _L, _L)]
        yg = cy_v[pl.ds(t * _L, _L)]
        cs = _corners(xg, yg, 64, 64)
        acc = [jnp.zeros((_L,), jnp.float32) for _ in range(20)]
        for (row, vf, w) in cs:
            rowb = row * 20
            for c in range(20):
                g = plsc.load_gather(mask_v, [rowb + c])
                acc[c] = acc[c] + (g * vf) * w
        m1 = acc[0]
        m2 = jnp.full((_L,), -jnp.inf, jnp.float32)
        for c in range(1, 20):
            v = acc[c]
            gt = v > m1
            m2 = jnp.where(gt, m1, jnp.maximum(m2, v))
            m1 = jnp.maximum(m1, v)
        unc_v[pl.ds(t * _L, _L)] = m2 - m1
        return _

    lax.fori_loop(0, (_NPTS // 4) // _L, step, 0)
    pltpu.sync_copy(unc_v, unc_hbm.at[img, pl.ds(base, _NPTS // 4)])


def _sample_unc(mask_t, cx, cy):
    mesh = plsc.VectorSubcoreMesh(core_axis_name="c", subcore_axis_name="s")
    f = pl.kernel(
        _unc_body,
        out_type=jax.ShapeDtypeStruct((8, _NPTS), jnp.float32),
        mesh=mesh,
        compiler_params=pltpu.CompilerParams(needs_layout_passes=False),
        scratch_types=[
            pltpu.VMEM((4096 * 20,), jnp.float32),
            pltpu.VMEM((_NPTS // 4,), jnp.float32),
            pltpu.VMEM((_NPTS // 4,), jnp.float32),
            pltpu.VMEM((_NPTS // 4,), jnp.float32),
        ],
    )
    return f(mask_t, cx, cy)


def _sort_body(unc_hbm, cx_hbm, cy_hbm, scx_hbm, scy_hbm,
               key_a, key_b, idx_a, idx_b, cnt, cx_v, cy_v, sx_v, sy_v):
    wid = _wid()
    lanes = lax.iota(jnp.int32, _L)
    lb = lanes * _CHUNK

    @pl.when(wid < 8)
    def _():
        img = wid
        pltpu.sync_copy(unc_hbm.at[img], key_a)
        pltpu.sync_copy(cx_hbm.at[img], cx_v)
        pltpu.sync_copy(cy_hbm.at[img], cy_v)

        minint = jnp.full((_L,), -2147483648, jnp.int32)
        izero = jnp.full((_L,), 0, jnp.int32)

        def keyify(t, c):
            u = key_a[pl.ds(t * _L, _L)]
            mono = jnp.where(u >= izero, u ^ minint, ~u)
            key_a[pl.ds(t * _L, _L)] = ~mono
            return c
        lax.fori_loop(0, _NPTS // _L, keyify, 0)

        def one_pass(src_k, src_i, dst_k, dst_i, shift, first, last):
            def zero(j, c):
                cnt[pl.ds(j * _L, _L)] = jnp.zeros((_L,), jnp.int32)
                return c
            lax.fori_loop(0, 256, zero, 0)

            shv = jnp.full((_L,), shift, jnp.int32)
            m255 = jnp.full((_L,), 255, jnp.int32)

            def hist(t, c):
                k = plsc.load_gather(src_k, [lb + t])
                d = lax.shift_right_logical(k, shv) & m255
                a = d * _L + lanes
                cv = plsc.load_gather(cnt, [a])
                plsc.store_scatter(cnt, [a], cv + 1)
                return c
            lax.fori_loop(0, _CHUNK, hist, 0)

            def scan(j, carry):
                v = cnt[pl.ds(j * _L, _L)]
                inc = plsc.cumsum(v)
                cnt[pl.ds(j * _L, _L)] = carry + (inc - v)
                return carry + jnp.sum(v)
            lax.fori_loop(0, 256, scan, jnp.int32(0))

            def perm(t, c):
                k = plsc.load_gather(src_k, [lb + t])
                if first:
                    v = lb + t
                else:
                    v = plsc.load_gather(src_i, [lb + t])
                d = lax.shift_right_logical(k, shv) & m255
                a = d * _L + lanes
                pos = plsc.load_gather(cnt, [a])
                plsc.store_scatter(cnt, [a], pos + 1)
                if last:
                    m = pos < _K
                    plsc.store_scatter(sx_v, [pos],
                                       plsc.load_gather(cx_v, [v]), mask=m)
                    plsc.store_scatter(sy_v, [pos],
                                       plsc.load_gather(cy_v, [v]), mask=m)
                else:
                    plsc.store_scatter(dst_k, [pos], k)
                    plsc.store_scatter(dst_i, [pos], v)
                return c
            lax.fori_loop(0, _CHUNK, perm, 0)

        one_pass(key_a, idx_a, key_b, idx_b, 0, True, False)
        one_pass(key_b, idx_b, key_a, idx_a, 8, False, False)
        one_pass(key_a, idx_a, key_b, idx_b, 16, False, False)
        one_pass(key_b, idx_b, key_a, idx_a, 24, False, True)

        pltpu.sync_copy(sx_v, scx_hbm.at[img])
        pltpu.sync_copy(sy_v, scy_hbm.at[img])


def _topk_coords(unc, cx, cy):
    unc = lax.bitcast_convert_type(unc, jnp.int32)
    mesh = plsc.VectorSubcoreMesh(core_axis_name="c", subcore_axis_name="s")
    f = pl.kernel(
        _sort_body,
        out_type=(jax.ShapeDtypeStruct((8, _K), jnp.float32),
                  jax.ShapeDtypeStruct((8, _K), jnp.float32)),
        mesh=mesh,
        compiler_params=pltpu.CompilerParams(needs_layout_passes=False),
        scratch_types=[
            pltpu.VMEM((_NPTS,), jnp.int32),
            pltpu.VMEM((_NPTS,), jnp.int32),
            pltpu.VMEM((_NPTS,), jnp.int32),
            pltpu.VMEM((_NPTS,), jnp.int32),
            pltpu.VMEM((256 * _L,), jnp.int32),
            pltpu.VMEM((_NPTS,), jnp.float32),
            pltpu.VMEM((_NPTS,), jnp.float32),
            pltpu.VMEM((_K,), jnp.float32),
            pltpu.VMEM((_K,), jnp.float32),
        ],
    )
    return f(unc, cx, cy)


def _coarse_body(mask_hbm, px_hbm, py_hbm, co_hbm, mask_v, px_v, py_v, co_v):
    wid = _wid()
    img = wid // 4
    pbase = (wid % 4) * _PT
    lanes = lax.iota(jnp.int32, _L)
    pltpu.sync_copy(mask_hbm.at[img], mask_v)
    pltpu.sync_copy(px_hbm.at[img, pl.ds(pbase, _PT)], px_v)
    pltpu.sync_copy(py_hbm.at[img, pl.ds(pbase, _PT)], py_v)

    def step(t, _):
        xg = px_v[pl.ds(t * _L, _L)]
        yg = py_v[pl.ds(t * _L, _L)]
        cs = _corners(xg, yg, 64, 64)
        pb20 = (t * _L + lanes) * 20
        for c in range(20):
            acc = jnp.zeros((_L,), jnp.float32)
            for (row, vf, w) in cs:
                g = plsc.load_gather(mask_v, [row * 20 + c])
                acc = acc + (g * vf) * w
            plsc.store_scatter(co_v, [pb20 + c], acc)
        return _

    lax.fori_loop(0, _PT // _L, step, 0)
    pltpu.sync_copy(co_v, co_hbm.at[img, pl.ds(pbase * 20, _PT * 20)])


def _sample_coarse(mask_t, px, py):
    mesh = plsc.VectorSubcoreMesh(core_axis_name="c", subcore_axis_name="s")
    f = pl.kernel(
        _coarse_body,
        out_type=jax.ShapeDtypeStruct((8, _P * 20), jnp.float32),
        mesh=mesh,
        compiler_params=pltpu.CompilerParams(needs_layout_passes=False),
        scratch_types=[
            pltpu.VMEM((4096 * 20,), jnp.float32),
            pltpu.VMEM((_PT,), jnp.float32),
            pltpu.VMEM((_PT,), jnp.float32),
            pltpu.VMEM((_PT * 20,), jnp.float32),
        ],
    )
    return f(mask_t, px, py)


def _fine_body(p2_hbm, px_hbm, py_hbm, fi_hbm, px_v, py_v, idx0, idx1,
               rows0, rows1, out0, out1, sem0, sem1, semo0, semo1):
    wid = _wid()
    img = wid // 4
    pbase = (wid % 4) * _PT
    pltpu.sync_copy(px_hbm.at[img, pl.ds(pbase, _PT)], px_v)
    pltpu.sync_copy(py_hbm.at[img, pl.ds(pbase, _PT)], py_v)
    gbase = img * 16384
    gdn = lax.GatherDimensionNumbers(offset_dims=(), collapsed_slice_dims=(0,),
                                     start_index_map=(0,))
    nchunk = _PT // _L

    def stage(t, idx_ref):
        xg = px_v[pl.ds(t * _L, _L)]
        yg = py_v[pl.ds(t * _L, _L)]
        cs = _corners(xg, yg, 128, 128)
        wv = []
        for k, (row, vf, w) in enumerate(cs):
            idx_ref[pl.ds(k * _L, _L)] = gbase + row
            wv.append(vf * w)
        return tuple(wv)

    def combine(t, rows_ref, out_ref, semo, wv):
        for p in range(_L):
            pidx = jnp.full((_L, 1), p, jnp.int32)
            wk = [lax.gather(wv[k], pidx, gdn, (1,),
                             mode=lax.GatherScatterMode.PROMISE_IN_BOUNDS)
                  for k in range(4)]
            for cg in range(256 // _L):
                acc = jnp.zeros((_L,), jnp.float32)
                for k in range(4):
                    r = rows_ref[k * _L + p, pl.ds(cg * _L, _L)]
                    acc = acc + r * wk[k]
                out_ref[pl.ds(p * 256 + cg * _L, _L)] = acc
        pltpu.async_copy(out_ref,
                         fi_hbm.at[img, pl.ds((pbase + t * _L) * 256,
                                              _L * 256)], semo)

    wv0 = stage(0, idx0)
    pltpu.async_copy(p2_hbm.at[idx0], rows0, sem0)

    def pair(u, wv0):
        t0 = u * 2
        t1 = u * 2 + 1
        wv1 = stage(t1, idx1)
        pltpu.async_copy(p2_hbm.at[idx1], rows1, sem1)
        pltpu.make_async_copy(p2_hbm.at[idx0], rows0, sem0).wait()

        @pl.when(u > 0)
        def _():
            pltpu.make_async_copy(
                out0, fi_hbm.at[img, pl.ds((pbase + (t0 - 2) * _L) * 256,
                                           _L * 256)], semo0).wait()
        combine(t0, rows0, out0, semo0, wv0)

        tn = jnp.minimum(t1 + 1, nchunk - 1)
        wvn = stage(tn, idx0)

        @pl.when(t1 + 1 < nchunk)
        def _():
            pltpu.async_copy(p2_hbm.at[idx0], rows0, sem0)
        pltpu.make_async_copy(p2_hbm.at[idx1], rows1, sem1).wait()

        @pl.when(u > 0)
        def _():
            pltpu.make_async_copy(
                out1, fi_hbm.at[img, pl.ds((pbase + (t1 - 2) * _L) * 256,
                                           _L * 256)], semo1).wait()
        combine(t1, rows1, out1, semo1, wv1)
        return wvn

    lax.fori_loop(0, nchunk // 2, pair, wv0)
    pltpu.make_async_copy(
        out0, fi_hbm.at[img, pl.ds((pbase + (nchunk - 2) * _L) * 256,
                                   _L * 256)], semo0).wait()
    pltpu.make_async_copy(
        out1, fi_hbm.at[img, pl.ds((pbase + (nchunk - 1) * _L) * 256,
                                   _L * 256)], semo1).wait()


def _sample_fine(p2f, px, py):
    mesh = plsc.VectorSubcoreMesh(core_axis_name="c", subcore_axis_name="s")
    f = pl.kernel(
        _fine_body,
        out_type=jax.ShapeDtypeStruct((8, _P * 256), jnp.float32),
        mesh=mesh,
        compiler_params=pltpu.CompilerParams(needs_layout_passes=False),
        scratch_types=[
            pltpu.VMEM((_PT,), jnp.float32),
            pltpu.VMEM((_PT,), jnp.float32),
            pltpu.VMEM((4 * _L,), jnp.int32),
            pltpu.VMEM((4 * _L,), jnp.int32),
            pltpu.VMEM((4 * _L, 256), jnp.float32),
            pltpu.VMEM((4 * _L, 256), jnp.float32),
            pltpu.VMEM((_L * 256,), jnp.float32),
            pltpu.VMEM((_L * 256,), jnp.float32),
            pltpu.SemaphoreType.DMA,
            pltpu.SemaphoreType.DMA,
            pltpu.SemaphoreType.DMA,
            pltpu.SemaphoreType.DMA,
        ],
    )
    return f(p2f, px, py)


def _head_body(c_ref, f_ref, wc1, wf1, b1, wc2, wf2, b2, wc3, wf3, b3,
               wcp, wfp, bp, o_ref):
    c = c_ref[...]
    f = f_ref[...]
    dot = functools.partial(jnp.dot, preferred_element_type=jnp.float32)
    h = jax.nn.relu(dot(c, wc1[...]) + dot(f, wf1[...]) + b1[...])
    h = jax.nn.relu(dot(h, wf2[...]) + dot(c, wc2[...]) + b2[...])
    h = jax.nn.relu(dot(h, wf3[...]) + dot(c, wc3[...]) + b3[...])
    o_ref[...] = dot(h, wfp[...]) + dot(c, wcp[...]) + bp[...]


def _point_head(coarse, fine, fc1_w, fc1_b, fc2_w, fc2_b, fc3_w, fc3_b,
                pred_w, pred_b):
    N, P, CC = coarse.shape
    BP = 2048
    wc1 = fc1_w[:, :CC].T
    wf1 = fc1_w[:, CC:].T
    wf2, wc2 = fc2_w[:, :256].T, fc2_w[:, 256:].T
    wf3, wc3 = fc3_w[:, :256].T, fc3_w[:, 256:].T
    wfp, wcp = pred_w[:, :256].T, pred_w[:, 256:].T
    b1, b2, b3, bp = (fc1_b[None, :], fc2_b[None, :], fc3_b[None, :],
                      pred_b[None, :])
    grid = (N, P // BP)
    wspec = lambda shape: pl.BlockSpec(shape, lambda n, p: (0, 0))
    return pl.pallas_call(
        _head_body,
        grid=grid,
        in_specs=[
            pl.BlockSpec((1, BP, CC), lambda n, p: (n, p, 0)),
            pl.BlockSpec((1, BP, 256), lambda n, p: (n, p, 0)),
            wspec(wc1.shape), wspec(wf1.shape), wspec(b1.shape),
            wspec(wc2.shape), wspec(wf2.shape), wspec(b2.shape),
            wspec(wc3.shape), wspec(wf3.shape), wspec(b3.shape),
            wspec(wcp.shape), wspec(wfp.shape), wspec(bp.shape),
        ],
        out_specs=pl.BlockSpec((1, BP, CC), lambda n, p: (n, p, 0)),
        out_shape=jax.ShapeDtypeStruct((N, P, CC), jnp.float32),
    )(coarse, fine, wc1, wf1, b1, wc2, wf2, b2, wc3, wf3, b3, wcp, wfp, bp)


def kernel(x, p2, masks, rand_coords, rand_extra, fc1_w, fc1_b, fc2_w, fc2_b,
           fc3_w, fc3_b, pred_w, pred_b):
    N = masks.shape[0]
    mask_t = jnp.transpose(masks.reshape(N, 20, 4096), (0, 2, 1))
    mask_t = mask_t.reshape(N, 4096 * 20)
    p2f = jnp.transpose(p2.reshape(N, 256, 16384), (0, 2, 1))
    p2f = p2f.reshape(N * 16384, 256)
    cx = rand_coords[..., 0]
    cy = rand_coords[..., 1]

    unc = _sample_unc(mask_t, cx, cy)
    sx, sy = _topk_coords(unc, cx, cy)
    px = jnp.concatenate([sx, rand_extra[..., 0]], axis=1)
    py = jnp.concatenate([sy, rand_extra[..., 1]], axis=1)
    point_coords = jnp.stack([px, py], axis=-1)

    coarse = _sample_coarse(mask_t, px, py).reshape(N, _P, 20)
    fine = _sample_fine(p2f, px, py).reshape(N, _P, 256)
    rend = _point_head(coarse, fine, fc1_w, fc1_b, fc2_w, fc2_b,
                       fc3_w, fc3_b, pred_w, pred_b)
    return (jnp.transpose(rend, (0, 2, 1)), point_coords)


# speedup vs baseline: 12.5994x; 1.1080x over previous
import functools
import jax
import jax.numpy as jnp
from jax import lax
from jax.experimental import pallas as pl
from jax.experimental.pallas import tpu as pltpu
from jax.experimental.pallas import tpu_sc as plsc

_NC, _NS, _L = 2, 16, 16
_NPTS = 12288
_K = 3072
_P = 4096
_PT = _P // 4
_CHUNK = _NPTS // _L


def _wid():
    return lax.axis_index("s") * _NC + lax.axis_index("c")


def _corners(xg, yg, H, W):
    ione = jnp.ones((_L,), jnp.int32)
    izero = jnp.zeros((_L,), jnp.int32)
    fone = jnp.full((_L,), 1.0, jnp.float32)
    fzero = jnp.zeros((_L,), jnp.float32)
    gx = 2.0 * xg - 1.0
    gy = 2.0 * yg - 1.0
    x = ((gx + 1.0) * jnp.float32(W) - 1.0) * 0.5
    y = ((gy + 1.0) * jnp.float32(H) - 1.0) * 0.5
    xt = x.astype(jnp.int32)
    x0i = xt - jnp.where(x < xt.astype(jnp.float32), ione, izero)
    yt = y.astype(jnp.int32)
    y0i = yt - jnp.where(y < yt.astype(jnp.float32), ione, izero)
    wx1 = x - x0i.astype(jnp.float32)
    wy1 = y - y0i.astype(jnp.float32)
    wx0 = 1.0 - wx1
    wy0 = 1.0 - wy1
    out = []
    for (dx, dy, wx, wy) in ((0, 0, wx0, wy0), (1, 0, wx1, wy0),
                             (0, 1, wx0, wy1), (1, 1, wx1, wy1)):
        ix = x0i + dx
        iy = y0i + dy
        valid = ((ix >= 0) & (ix <= W - 1) & (iy >= 0) & (iy <= H - 1))
        vf = jnp.where(valid, fone, fzero)
        w = wx * wy
        ixc = jnp.minimum(jnp.maximum(ix, izero), W - 1)
        iyc = jnp.minimum(jnp.maximum(iy, izero), H - 1)
        out.append((iyc * W + ixc, vf, w))
    return out


def _unc_body(mask_hbm, cx_hbm, cy_hbm, unc_hbm, mask_v, cx_v, cy_v, unc_v):
    wid = _wid()
    img = wid // 4
    base = (wid % 4) * (_NPTS // 4)
    pltpu.sync_copy(mask_hbm.at[img], mask_v)
    pltpu.sync_copy(cx_hbm.at[img, pl.ds(base, _NPTS // 4)], cx_v)
    pltpu.sync_copy(cy_hbm.at[img, pl.ds(base, _NPTS // 4)], cy_v)

    def step(t, _):
        xg = cx_v[pl.ds(t * _L, _L)]
        yg = cy_v[pl.ds(t * _L, _L)]
        cs = _corners(xg, yg, 64, 64)
        acc = [jnp.zeros((_L,), jnp.float32) for _ in range(20)]
        for (row, vf, w) in cs:
            rowb = row * 20
            for c in range(20):
                g = plsc.load_gather(mask_v, [rowb + c])
                acc[c] = acc[c] + (g * vf) * w
        m1 = acc[0]
        m2 = jnp.full((_L,), -jnp.inf, jnp.float32)
        for c in range(1, 20):
            v = acc[c]
            gt = v > m1
            m2 = jnp.where(gt, m1, jnp.maximum(m2, v))
            m1 = jnp.maximum(m1, v)
        unc_v[pl.ds(t * _L, _L)] = m2 - m1
        return _

    lax.fori_loop(0, (_NPTS // 4) // _L, step, 0)
    pltpu.sync_copy(unc_v, unc_hbm.at[img, pl.ds(base, _NPTS // 4)])


def _sample_unc(mask_t, cx, cy):
    mesh = plsc.VectorSubcoreMesh(core_axis_name="c", subcore_axis_name="s")
    f = pl.kernel(
        _unc_body,
        out_type=jax.ShapeDtypeStruct((8, _NPTS), jnp.float32),
        mesh=mesh,
        compiler_params=pltpu.CompilerParams(needs_layout_passes=False),
        scratch_types=[
            pltpu.VMEM((4096 * 20,), jnp.float32),
            pltpu.VMEM((_NPTS // 4,), jnp.float32),
            pltpu.VMEM((_NPTS // 4,), jnp.float32),
            pltpu.VMEM((_NPTS // 4,), jnp.float32),
        ],
    )
    return f(mask_t, cx, cy)


def _sort_body(unc_hbm, cx_hbm, cy_hbm, scx_hbm, scy_hbm,
               key_a, key_b, idx_a, idx_b, cnt, cx_v, cy_v, sx_v, sy_v):
    wid = _wid()
    lanes = lax.iota(jnp.int32, _L)
    lb = lanes * _CHUNK

    @pl.when(wid < 8)
    def _():
        img = wid
        pltpu.sync_copy(unc_hbm.at[img], key_a)
        pltpu.sync_copy(cx_hbm.at[img], cx_v)
        pltpu.sync_copy(cy_hbm.at[img], cy_v)

        minint = jnp.full((_L,), -2147483648, jnp.int32)
        izero = jnp.full((_L,), 0, jnp.int32)
        ione = jnp.full((_L,), 1, jnp.int32)
        sh24 = jnp.full((_L,), 24, jnp.int32)
        m255 = jnp.full((_L,), 255, jnp.int32)

        def keyify(t, c):
            u = key_a[pl.ds(t * _L, _L)]
            mono = jnp.where(u >= izero, u ^ minint, ~u)
            key_a[pl.ds(t * _L, _L)] = ~mono
            return c
        lax.fori_loop(0, _NPTS // _L, keyify, 0)

        def zero(j, c):
            cnt[pl.ds(j * _L, _L)] = jnp.zeros((_L,), jnp.int32)
            return c
        lax.fori_loop(0, 256, zero, 0)

        def hist1(t, c):
            k = plsc.load_gather(key_a, [lb + t])
            d = lax.shift_right_logical(k, sh24) & m255
            a = d * _L + lanes
            cv = plsc.load_gather(cnt, [a])
            plsc.store_scatter(cnt, [a], cv + 1)
            return c
        lax.fori_loop(0, _CHUNK, hist1, 0)

        def find(j, carry):
            run, dstar, msel = carry
            tj = jnp.sum(cnt[pl.ds(j * _L, _L)])
            run2 = run + tj
            dstar = jnp.where(run < _K, j, dstar)
            msel = jnp.where(run < _K, run2, msel)
            return (run2, dstar, msel)
        _run, dstar, m = lax.fori_loop(
            0, 256, find, (jnp.int32(0), jnp.int32(0), jnp.int32(0)))

        def kcount(t, kc):
            k = plsc.load_gather(key_a, [lb + t])
            d = lax.shift_right_logical(k, sh24) & m255
            return kc + jnp.where(d <= dstar, ione, izero)
        kcnt = lax.fori_loop(0, _CHUNK, kcount, jnp.zeros((_L,), jnp.int32))
        koff = plsc.cumsum(kcnt) - kcnt

        def compact(t, rk):
            k = plsc.load_gather(key_a, [lb + t])
            d = lax.shift_right_logical(k, sh24) & m255
            keep = d <= dstar
            pos = koff + rk
            plsc.store_scatter(key_b, [pos], k, mask=keep)
            plsc.store_scatter(idx_b, [pos], lb + t, mask=keep)
            return rk + jnp.where(keep, ione, izero)
        lax.fori_loop(0, _CHUNK, compact, jnp.zeros((_L,), jnp.int32))

        cpl = (m + _L - 1) // _L
        lbm = lanes * cpl
        mend = jnp.full((_L,), 0, jnp.int32) + m

        def one_pass(src_k, src_i, dst_k, dst_i, shift, last):
            lax.fori_loop(0, 256, zero, 0)
            shv = jnp.full((_L,), shift, jnp.int32)

            def hist(t, c):
                e = lbm + t
                valid = e < mend
                k = plsc.load_gather(src_k, [jnp.minimum(e, mend - 1)])
                d = lax.shift_right_logical(k, shv) & m255
                a = d * _L + lanes
                cv = plsc.load_gather(cnt, [a])
                plsc.store_scatter(cnt, [a],
                                   cv + jnp.where(valid, ione, izero))
                return c
            lax.fori_loop(0, cpl, hist, 0)

            def scan(j, carry):
                v = cnt[pl.ds(j * _L, _L)]
                inc = plsc.cumsum(v)
                cnt[pl.ds(j * _L, _L)] = carry + (inc - v)
                return carry + jnp.sum(v)
            lax.fori_loop(0, 256, scan, jnp.int32(0))

            def perm(t, c):
                e = jnp.minimum(lbm + t, mend - 1)
                valid = (lbm + t) < mend
                k = plsc.load_gather(src_k, [e])
                v = plsc.load_gather(src_i, [e])
                d = lax.shift_right_logical(k, shv) & m255
                a = d * _L + lanes
                pos = plsc.load_gather(cnt, [a])
                plsc.store_scatter(cnt, [a],
                                   pos + jnp.where(valid, ione, izero))
                if last:
                    msk = valid & (pos < _K)
                    plsc.store_scatter(sx_v, [pos],
                                       plsc.load_gather(cx_v, [v]), mask=msk)
                    plsc.store_scatter(sy_v, [pos],
                                       plsc.load_gather(cy_v, [v]), mask=msk)
                else:
                    plsc.store_scatter(dst_k, [pos], k, mask=valid)
                    plsc.store_scatter(dst_i, [pos], v, mask=valid)
                return c
            lax.fori_loop(0, cpl, perm, 0)

        one_pass(key_b, idx_b, key_a, idx_a, 0, False)
        one_pass(key_a, idx_a, key_b, idx_b, 8, False)
        one_pass(key_b, idx_b, key_a, idx_a, 16, False)
        one_pass(key_a, idx_a, key_b, idx_b, 24, True)

        pltpu.sync_copy(sx_v, scx_hbm.at[img])
        pltpu.sync_copy(sy_v, scy_hbm.at[img])


def _topk_coords(unc, cx, cy):
    unc = lax.bitcast_convert_type(unc, jnp.int32)
    mesh = plsc.VectorSubcoreMesh(core_axis_name="c", subcore_axis_name="s")
    f = pl.kernel(
        _sort_body,
        out_type=(jax.ShapeDtypeStruct((8, _K), jnp.float32),
                  jax.ShapeDtypeStruct((8, _K), jnp.float32)),
        mesh=mesh,
        compiler_params=pltpu.CompilerParams(needs_layout_passes=False),
        scratch_types=[
            pltpu.VMEM((_NPTS,), jnp.int32),
            pltpu.VMEM((_NPTS,), jnp.int32),
            pltpu.VMEM((_NPTS,), jnp.int32),
            pltpu.VMEM((_NPTS,), jnp.int32),
            pltpu.VMEM((256 * _L,), jnp.int32),
            pltpu.VMEM((_NPTS,), jnp.float32),
            pltpu.VMEM((_NPTS,), jnp.float32),
            pltpu.VMEM((_K,), jnp.float32),
            pltpu.VMEM((_K,), jnp.float32),
        ],
    )
    return f(unc, cx, cy)


def _coarse_body(mask_hbm, px_hbm, py_hbm, co_hbm, mask_v, px_v, py_v, co_v):
    wid = _wid()
    img = wid // 4
    pbase = (wid % 4) * _PT
    lanes = lax.iota(jnp.int32, _L)
    pltpu.sync_copy(mask_hbm.at[img], mask_v)
    pltpu.sync_copy(px_hbm.at[img, pl.ds(pbase, _PT)], px_v)
    pltpu.sync_copy(py_hbm.at[img, pl.ds(pbase, _PT)], py_v)

    def step(t, _):
        xg = px_v[pl.ds(t * _L, _L)]
        yg = py_v[pl.ds(t * _L, _L)]
        cs = _corners(xg, yg, 64, 64)
        pb20 = (t * _L + lanes) * 20
        for c in range(20):
            acc = jnp.zeros((_L,), jnp.float32)
            for (row, vf, w) in cs:
                g = plsc.load_gather(mask_v, [row * 20 + c])
                acc = acc + (g * vf) * w
            plsc.store_scatter(co_v, [pb20 + c], acc)
        return _

    lax.fori_loop(0, _PT // _L, step, 0)
    pltpu.sync_copy(co_v, co_hbm.at[img, pl.ds(pbase * 20, _PT * 20)])


def _sample_coarse(mask_t, px, py):
    mesh = plsc.VectorSubcoreMesh(core_axis_name="c", subcore_axis_name="s")
    f = pl.kernel(
        _coarse_body,
        out_type=jax.ShapeDtypeStruct((8, _P * 20), jnp.float32),
        mesh=mesh,
        compiler_params=pltpu.CompilerParams(needs_layout_passes=False),
        scratch_types=[
            pltpu.VMEM((4096 * 20,), jnp.float32),
            pltpu.VMEM((_PT,), jnp.float32),
            pltpu.VMEM((_PT,), jnp.float32),
            pltpu.VMEM((_PT * 20,), jnp.float32),
        ],
    )
    return f(mask_t, px, py)


def _fine_body(p2_hbm, px_hbm, py_hbm, fi_hbm, px_v, py_v, idx0, idx1,
               rows0, rows1, out0, out1, sem0, sem1, semo0, semo1):
    wid = _wid()
    img = wid // 4
    pbase = (wid % 4) * _PT
    pltpu.sync_copy(px_hbm.at[img, pl.ds(pbase, _PT)], px_v)
    pltpu.sync_copy(py_hbm.at[img, pl.ds(pbase, _PT)], py_v)
    gbase = img * 16384
    gdn = lax.GatherDimensionNumbers(offset_dims=(), collapsed_slice_dims=(0,),
                                     start_index_map=(0,))
    nchunk = _PT // _L

    def stage(t, idx_ref):
        xg = px_v[pl.ds(t * _L, _L)]
        yg = py_v[pl.ds(t * _L, _L)]
        cs = _corners(xg, yg, 128, 128)
        wv = []
        for k, (row, vf, w) in enumerate(cs):
            idx_ref[pl.ds(k * _L, _L)] = gbase + row
            wv.append(vf * w)
        return tuple(wv)

    def combine(t, rows_ref, out_ref, semo, wv):
        for p in range(_L):
            pidx = jnp.full((_L, 1), p, jnp.int32)
            wk = [lax.gather(wv[k], pidx, gdn, (1,),
                             mode=lax.GatherScatterMode.PROMISE_IN_BOUNDS)
                  for k in range(4)]
            for cg in range(256 // _L):
                acc = jnp.zeros((_L,), jnp.float32)
                for k in range(4):
                    r = rows_ref[k * _L + p, pl.ds(cg * _L, _L)]
                    acc = acc + r * wk[k]
                out_ref[pl.ds(p * 256 + cg * _L, _L)] = acc
        pltpu.async_copy(out_ref,
                         fi_hbm.at[img, pl.ds((pbase + t * _L) * 256,
                                              _L * 256)], semo)

    wv0 = stage(0, idx0)
    pltpu.async_copy(p2_hbm.at[idx0], rows0, sem0)

    def pair(u, wv0):
        t0 = u * 2
        t1 = u * 2 + 1
        wv1 = stage(t1, idx1)
        pltpu.async_copy(p2_hbm.at[idx1], rows1, sem1)
        pltpu.make_async_copy(p2_hbm.at[idx0], rows0, sem0).wait()

        @pl.when(u > 0)
        def _():
            pltpu.make_async_copy(
                out0, fi_hbm.at[img, pl.ds((pbase + (t0 - 2) * _L) * 256,
                                           _L * 256)], semo0).wait()
        combine(t0, rows0, out0, semo0, wv0)

        tn = jnp.minimum(t1 + 1, nchunk - 1)
        wvn = stage(tn, idx0)

        @pl.when(t1 + 1 < nchunk)
        def _():
            pltpu.async_copy(p2_hbm.at[idx0], rows0, sem0)
        pltpu.make_async_copy(p2_hbm.at[idx1], rows1, sem1).wait()

        @pl.when(u > 0)
        def _():
            pltpu.make_async_copy(
                out1, fi_hbm.at[img, pl.ds((pbase + (t1 - 2) * _L) * 256,
                                           _L * 256)], semo1).wait()
        combine(t1, rows1, out1, semo1, wv1)
        return wvn

    lax.fori_loop(0, nchunk // 2, pair, wv0)
    pltpu.make_async_copy(
        out0, fi_hbm.at[img, pl.ds((pbase + (nchunk - 2) * _L) * 256,
                                   _L * 256)], semo0).wait()
    pltpu.make_async_copy(
        out1, fi_hbm.at[img, pl.ds((pbase + (nchunk - 1) * _L) * 256,
                                   _L * 256)], semo1).wait()


def _sample_fine(p2f, px, py):
    mesh = plsc.VectorSubcoreMesh(core_axis_name="c", subcore_axis_name="s")
    f = pl.kernel(
        _fine_body,
        out_type=jax.ShapeDtypeStruct((8, _P * 256), jnp.float32),
        mesh=mesh,
        compiler_params=pltpu.CompilerParams(needs_layout_passes=False),
        scratch_types=[
            pltpu.VMEM((_PT,), jnp.float32),
            pltpu.VMEM((_PT,), jnp.float32),
            pltpu.VMEM((4 * _L,), jnp.int32),
            pltpu.VMEM((4 * _L,), jnp.int32),
            pltpu.VMEM((4 * _L, 256), jnp.float32),
            pltpu.VMEM((4 * _L, 256), jnp.float32),
            pltpu.VMEM((_L * 256,), jnp.float32),
            pltpu.VMEM((_L * 256,), jnp.float32),
            pltpu.SemaphoreType.DMA,
            pltpu.SemaphoreType.DMA,
            pltpu.SemaphoreType.DMA,
            pltpu.SemaphoreType.DMA,
        ],
    )
    return f(p2f, px, py)


def _head_body(c_ref, f_ref, wc1, wf1, b1, wc2, wf2, b2, wc3, wf3, b3,
               wcp, wfp, bp, o_ref):
    c = c_ref[...]
    f = f_ref[...]
    dot = functools.partial(jnp.dot, preferred_element_type=jnp.float32)
    h = jax.nn.relu(dot(c, wc1[...]) + dot(f, wf1[...]) + b1[...])
    h = jax.nn.relu(dot(h, wf2[...]) + dot(c, wc2[...]) + b2[...])
    h = jax.nn.relu(dot(h, wf3[...]) + dot(c, wc3[...]) + b3[...])
    o_ref[...] = dot(h, wfp[...]) + dot(c, wcp[...]) + bp[...]


def _point_head(coarse, fine, fc1_w, fc1_b, fc2_w, fc2_b, fc3_w, fc3_b,
                pred_w, pred_b):
    N, P, CC = coarse.shape
    BP = 2048
    wc1 = fc1_w[:, :CC].T
    wf1 = fc1_w[:, CC:].T
    wf2, wc2 = fc2_w[:, :256].T, fc2_w[:, 256:].T
    wf3, wc3 = fc3_w[:, :256].T, fc3_w[:, 256:].T
    wfp, wcp = pred_w[:, :256].T, pred_w[:, 256:].T
    b1, b2, b3, bp = (fc1_b[None, :], fc2_b[None, :], fc3_b[None, :],
                      pred_b[None, :])
    grid = (N, P // BP)
    wspec = lambda shape: pl.BlockSpec(shape, lambda n, p: (0, 0))
    return pl.pallas_call(
        _head_body,
        grid=grid,
        in_specs=[
            pl.BlockSpec((1, BP, CC), lambda n, p: (n, p, 0)),
            pl.BlockSpec((1, BP, 256), lambda n, p: (n, p, 0)),
            wspec(wc1.shape), wspec(wf1.shape), wspec(b1.shape),
            wspec(wc2.shape), wspec(wf2.shape), wspec(b2.shape),
            wspec(wc3.shape), wspec(wf3.shape), wspec(b3.shape),
            wspec(wcp.shape), wspec(wfp.shape), wspec(bp.shape),
        ],
        out_specs=pl.BlockSpec((1, BP, CC), lambda n, p: (n, p, 0)),
        out_shape=jax.ShapeDtypeStruct((N, P, CC), jnp.float32),
    )(coarse, fine, wc1, wf1, b1, wc2, wf2, b2, wc3, wf3, b3, wcp, wfp, bp)


def kernel(x, p2, masks, rand_coords, rand_extra, fc1_w, fc1_b, fc2_w, fc2_b,
           fc3_w, fc3_b, pred_w, pred_b):
    N = masks.shape[0]
    mask_t = jnp.transpose(masks.reshape(N, 20, 4096), (0, 2, 1))
    mask_t = mask_t.reshape(N, 4096 * 20)
    p2f = jnp.transpose(p2.reshape(N, 256, 16384), (0, 2, 1))
    p2f = p2f.reshape(N * 16384, 256)
    cx = rand_coords[..., 0]
    cy = rand_coords[..., 1]

    unc = _sample_unc(mask_t, cx, cy)
    sx, sy = _topk_coords(unc, cx, cy)
    px = jnp.concatenate([sx, rand_extra[..., 0]], axis=1)
    py = jnp.concatenate([sy, rand_extra[..., 1]], axis=1)
    point_coords = jnp.stack([px, py], axis=-1)

    coarse = _sample_coarse(mask_t, px, py).reshape(N, _P, 20)
    fine = _sample_fine(p2f, px, py).reshape(N, _P, 256)
    rend = _point_head(coarse, fine, fc1_w, fc1_b, fc2_w, fc2_b,
                       fc3_w, fc3_b, pred_w, pred_b)
    return (jnp.transpose(rend, (0, 2, 1)), point_coords)
